# baseline (speedup 1.0000x reference)
import jax
import jax.numpy as jnp
from jax import lax
from jax.experimental import pallas as pl
from jax.experimental.pallas import tpu as pltpu
from jax.experimental.pallas import tpu_sc as plsc

_NC = 2
_NS = 16
_NW = _NC * _NS
_L = 16
_K = 80
_DW = 16
_FRAC0 = 0.40


def _sc_mesh():
    return plsc.VectorSubcoreMesh(core_axis_name="c", subcore_axis_name="s",
                                  num_cores=_NC, num_subcores=_NS)


def _scale_rows(rows, ewb_v, d):
    def grp(g, carry):
        for l in range(_L):
            e = g * _L + l
            w = ewb_v[e, :]
            for j in range(d // _L):
                sl = pl.ds(j * _L, _L)
                rows[e, sl] = rows[e, sl] * w
        return carry

    lax.fori_loop(0, _K // _L, grp, 0)


def _fill_rows(rows, ewb_v):
    def grp(g, carry):
        for l in range(_L):
            e = g * _L + l
            w = ewb_v[e, :]
            for j in range(128 // _L):
                rows[e, pl.ds(j * _L, _L)] = w
        return carry

    lax.fori_loop(0, _K // _L, grp, 0)


def _deg_kernel(n_pad, ne0, ne1, rpt):
    def body(idx_hbm, ewb_hbm, zero_hbm, out_hbm,
             idx0, idx1, ewb0, ewb1, rows0, rows1, acc,
             si0, si1, se0, se1, ss0):
        c = lax.axis_index("c")
        s = lax.axis_index("s")
        wid = c * _NS + s
        row0 = s * rpt
        pltpu.sync_copy(zero_hbm.at[pl.ds(row0, rpt)], acc.at[pl.ds(row0, rpt)])
        plsc.subcore_barrier()

        pltpu.sync_copy(idx_hbm.at[wid, 0], idx0)
        pltpu.sync_copy(ewb_hbm.at[wid, 0], ewb0)

        def pair(i, carry):
            a = 2 * i
            b = a + 1
            pltpu.async_copy(idx_hbm.at[wid, b], idx1, si1)
            pltpu.async_copy(ewb_hbm.at[wid, b], ewb1, se1)
            _fill_rows(rows0, ewb0)
            pltpu.make_async_copy(idx_hbm.at[wid, b], idx1, si1).wait()
            pltpu.make_async_copy(ewb_hbm.at[wid, b], ewb1, se1).wait()
            pltpu.async_copy(rows0, acc.at[idx0.at[1]], ss0, add=True)
            _fill_rows(rows1, ewb1)
            pltpu.make_async_copy(rows0, acc.at[idx0.at[1]], ss0).wait()
            pltpu.async_copy(idx_hbm.at[wid, a + 2], idx0, si0)
            pltpu.async_copy(ewb_hbm.at[wid, a + 2], ewb0, se0)
            pltpu.sync_copy(rows1, acc.at[idx1.at[1]], add=True)
            pltpu.make_async_copy(idx_hbm.at[wid, a + 2], idx0, si0).wait()
            pltpu.make_async_copy(ewb_hbm.at[wid, a + 2], ewb0, se0).wait()
            return carry

        trip = lax.select(c == 0, ne0 // 2, ne1 // 2)
        lax.fori_loop(0, trip, pair, 0)
        plsc.subcore_barrier()
        pltpu.sync_copy(acc.at[pl.ds(row0, rpt)], out_hbm.at[c, pl.ds(row0, rpt)])

    return pl.kernel(
        body,
        out_type=jax.ShapeDtypeStruct((_NC, n_pad, 128), jnp.float32),
        mesh=_sc_mesh(),
        scratch_types=[
            pltpu.VMEM((2, _K), jnp.int32),
            pltpu.VMEM((2, _K), jnp.int32),
            pltpu.VMEM((_K, _DW), jnp.float32),
            pltpu.VMEM((_K, _DW), jnp.float32),
            pltpu.VMEM((_K, 128), jnp.float32),
            pltpu.VMEM((_K, 128), jnp.float32),
            pltpu.VMEM_SHARED((n_pad, 128), jnp.float32),
            pltpu.SemaphoreType.DMA,
            pltpu.SemaphoreType.DMA,
            pltpu.SemaphoreType.DMA,
            pltpu.SemaphoreType.DMA,
            pltpu.SemaphoreType.DMA,
        ],
    )


def _agg_kernel(d, n_pad, ne0, ne1, rpt):
    def body(h_hbm, idx_hbm, ewb_hbm, zero_hbm, out_hbm,
             idx0, idx1, ewb0, ewb1, rows0, rows1, acc,
             sg0, sg1, si0, si1, se0, se1, ss0):
        c = lax.axis_index("c")
        s = lax.axis_index("s")
        wid = c * _NS + s
        row0 = s * rpt
        pltpu.sync_copy(zero_hbm.at[pl.ds(row0, rpt)], acc.at[pl.ds(row0, rpt)])
        plsc.subcore_barrier()

        pltpu.sync_copy(idx_hbm.at[wid, 0], idx0)
        pltpu.sync_copy(ewb_hbm.at[wid, 0], ewb0)
        pltpu.async_copy(h_hbm.at[idx0.at[0]], rows0, sg0)

        def pair(i, carry):
            a = 2 * i
            b = a + 1
            pltpu.async_copy(idx_hbm.at[wid, b], idx1, si1)
            pltpu.async_copy(ewb_hbm.at[wid, b], ewb1, se1)
            pltpu.make_async_copy(h_hbm.at[idx0.at[0]], rows0, sg0).wait()
            _scale_rows(rows0, ewb0, d)
            pltpu.make_async_copy(idx_hbm.at[wid, b], idx1, si1).wait()
            pltpu.make_async_copy(ewb_hbm.at[wid, b], ewb1, se1).wait()
            pltpu.async_copy(h_hbm.at[idx1.at[0]], rows1, sg1)
            pltpu.async_copy(rows0, acc.at[idx0.at[1]], ss0, add=True)
            pltpu.make_async_copy(h_hbm.at[idx1.at[0]], rows1, sg1).wait()
            _scale_rows(rows1, ewb1, d)
            pltpu.make_async_copy(rows0, acc.at[idx0.at[1]], ss0).wait()
            pltpu.async_copy(idx_hbm.at[wid, a + 2], idx0, si0)
            pltpu.async_copy(ewb_hbm.at[wid, a + 2], ewb0, se0)
            pltpu.sync_copy(rows1, acc.at[idx1.at[1]], add=True)
            pltpu.make_async_copy(idx_hbm.at[wid, a + 2], idx0, si0).wait()
            pltpu.make_async_copy(ewb_hbm.at[wid, a + 2], ewb0, se0).wait()
            pltpu.async_copy(h_hbm.at[idx0.at[0]], rows0, sg0)
            return carry

        trip = lax.select(c == 0, ne0 // 2, ne1 // 2)
        lax.fori_loop(0, trip, pair, 0)
        pltpu.make_async_copy(h_hbm.at[idx0.at[0]], rows0, sg0).wait()
        plsc.subcore_barrier()
        pltpu.sync_copy(acc.at[pl.ds(row0, rpt)], out_hbm.at[c, pl.ds(row0, rpt)])

    return pl.kernel(
        body,
        out_type=jax.ShapeDtypeStruct((_NC, n_pad, d), jnp.float32),
        mesh=_sc_mesh(),
        scratch_types=[
            pltpu.VMEM((2, _K), jnp.int32),
            pltpu.VMEM((2, _K), jnp.int32),
            pltpu.VMEM((_K, _DW), jnp.float32),
            pltpu.VMEM((_K, _DW), jnp.float32),
            pltpu.VMEM((_K, d), jnp.float32),
            pltpu.VMEM((_K, d), jnp.float32),
            pltpu.VMEM_SHARED((n_pad, d), jnp.float32),
            pltpu.SemaphoreType.DMA,
            pltpu.SemaphoreType.DMA,
            pltpu.SemaphoreType.DMA,
            pltpu.SemaphoreType.DMA,
            pltpu.SemaphoreType.DMA,
            pltpu.SemaphoreType.DMA,
            pltpu.SemaphoreType.DMA,
        ],
    )


def _tc_mm(n, d):
    def body(x_ref, w_ref, out_ref):
        out_ref[...] = jnp.dot(x_ref[...], w_ref[...],
                               preferred_element_type=jnp.float32)

    return pl.pallas_call(
        body,
        out_shape=[jax.ShapeDtypeStruct((n, d), jnp.float32)],
    )


def _tc_prep(n, d):
    def body(m_ref, dp_ref, dis_ref, hp_ref):
        deg = dp_ref[0, :n, 0:1] + dp_ref[1, :n, 0:1] + 1.0
        dis = lax.rsqrt(deg)
        dis_ref[...] = dis
        hp_ref[...] = m_ref[...] * dis

    return pl.pallas_call(
        body,
        out_shape=[
            jax.ShapeDtypeStruct((n, 1), jnp.float32),
            jax.ShapeDtypeStruct((n, d), jnp.float32),
        ],
    )


def _tc_mid(n, d):
    def body(ap_ref, hp_ref, dis_ref, b_ref, w_ref, out_ref):
        dis = dis_ref[...]
        pre = (ap_ref[0, :n, :] + ap_ref[1, :n, :] + hp_ref[...]) * dis + b_ref[...]
        h = jnp.maximum(pre, 0.0)
        out_ref[...] = jnp.dot(h, w_ref[...], preferred_element_type=jnp.float32) * dis

    return pl.pallas_call(
        body,
        out_shape=[jax.ShapeDtypeStruct((n, d), jnp.float32)],
    )


def _tc_head(n, d, nb, dout):
    def body(ap_ref, hp_ref, dis_ref, b_ref, batch_ref, bs_ref, u_ref,
             wha_ref, whb_ref, bh1_ref, wh2_ref, bh2_ref, out_ref):
        dis = dis_ref[...]
        h2 = jnp.maximum(
            (ap_ref[0, :n, :] + ap_ref[1, :n, :] + hp_ref[...]) * dis + b_ref[...],
            0.0)
        seg = lax.broadcasted_iota(jnp.int32, (nb, n), 0)
        onehot = (seg == batch_ref[...]).astype(jnp.float32)
        sums = jnp.dot(onehot, h2, preferred_element_type=jnp.float32)
        cnts = jnp.sum(onehot, axis=1, keepdims=True)
        gem = sums / jnp.maximum(cnts, 1.0)
        valid = lax.broadcasted_iota(jnp.int32, (nb, 1), 0) < bs_ref[0]
        gem = jnp.where(valid, gem, 0.0)
        z = jnp.maximum(
            jnp.dot(gem, wha_ref[...], preferred_element_type=jnp.float32)
            + jnp.dot(u_ref[...], whb_ref[...], preferred_element_type=jnp.float32)
            + bh1_ref[...], 0.0)
        out_ref[...] = (jnp.dot(z, wh2_ref[...], preferred_element_type=jnp.float32)
                        + bh2_ref[...])

    nargs = 12
    specs = [pl.BlockSpec(memory_space=pltpu.VMEM) for _ in range(nargs)]
    specs[5] = pl.BlockSpec(memory_space=pltpu.SMEM)
    return pl.pallas_call(
        body,
        in_specs=specs,
        out_specs=[pl.BlockSpec(memory_space=pltpu.VMEM)],
        out_shape=[jax.ShapeDtypeStruct((nb, dout), jnp.float32)],
    )


def kernel(x, edge_index, u, edge_weight, batch, batch_size,
           W1, b1, W2, b2, Wh1, bh1, Wh2, bh2):
    n, _ = x.shape
    d = W1.shape[1]
    e = edge_weight.shape[0]
    nb, dg = u.shape
    dout = Wh2.shape[1]

    grp0 = _NS * _K
    ne0 = -(-int(e * _FRAC0) // grp0)
    ne0 = ne0 + (ne0 % 2)
    e0 = min(e, grp0 * ne0)
    ne1 = -(-(e - e0) // grp0)
    ne1 = ne1 + (ne1 % 2)
    nm = max(ne0, ne1) + 2
    rpt = (-(-n // _NS) + 7) // 8 * 8
    n_pad = rpt * _NS

    def split(arr, dtype):
        a0 = arr[:e0].reshape(_NS, ne0, _K)
        a0 = jnp.pad(a0, ((0, 0), (0, nm - ne0), (0, 0)))
        a1 = jnp.pad(arr[e0:], (0, grp0 * ne1 - (e - e0)))
        a1 = jnp.pad(a1.reshape(_NS, ne1, _K), ((0, 0), (0, nm - ne1), (0, 0)))
        return jnp.concatenate([a0, a1], axis=0).astype(dtype)

    srcp = split(edge_index[0], jnp.int32)[:, :, None, :]
    dstp = split(edge_index[1], jnp.int32)[:, :, None, :]
    idxp = jnp.concatenate([srcp, dstp], axis=2)
    ewb = jnp.broadcast_to(
        split(edge_weight, jnp.float32)[:, :, :, None],
        (_NW, nm, _K, _DW))
    zero_w = jnp.zeros((n_pad, d), jnp.float32)

    (m1,) = _tc_mm(n, d)(x, W1)
    degp = _deg_kernel(n_pad, ne0, ne1, rpt)(idxp, ewb, zero_w)
    dis, h1p = _tc_prep(n, d)(m1, degp)

    agg = _agg_kernel(d, n_pad, ne0, ne1, rpt)
    a1 = agg(h1p, idxp, ewb, zero_w)
    (h2p,) = _tc_mid(n, d)(a1, h1p, dis, b1.reshape(1, d), W2)
    a2 = agg(h2p, idxp, ewb, zero_w)

    (out,) = _tc_head(n, d, nb, dout)(
        a2, h2p, dis, b2.reshape(1, d),
        batch.reshape(1, n).astype(jnp.int32),
        jnp.asarray(batch_size, jnp.int32).reshape(1),
        u, Wh1[:d], Wh1[d:], bh1.reshape(1, d), Wh2, bh2.reshape(1, dout))
    return out


# speedup vs baseline: 1.0783x; 1.0783x over previous
import jax
import jax.numpy as jnp
from jax import lax
from jax.experimental import pallas as pl
from jax.experimental.pallas import tpu as pltpu
from jax.experimental.pallas import tpu_sc as plsc

_NC = 2
_NS = 16
_NW = _NC * _NS
_L = 16
_K = 80
_DW = 16
_FRAC0 = 0.48


def _sc_mesh():
    return plsc.VectorSubcoreMesh(core_axis_name="c", subcore_axis_name="s",
                                  num_cores=_NC, num_subcores=_NS)


def _scale_rows(rows, ewb_v, d):
    def grp(g, carry):
        for l in range(_L):
            e = g * _L + l
            w = ewb_v[e, :]
            for j in range(d // _L):
                sl = pl.ds(j * _L, _L)
                rows[e, sl] = rows[e, sl] * w
        return carry

    lax.fori_loop(0, _K // _L, grp, 0)


def _fill_rows(rows, ewb_v):
    def grp(g, carry):
        for l in range(_L):
            e = g * _L + l
            w = ewb_v[e, :]
            for j in range(128 // _L):
                rows[e, pl.ds(j * _L, _L)] = w
        return carry

    lax.fori_loop(0, _K // _L, grp, 0)


def _deg_kernel(n_pad, ne0, ne1, rpt):
    def body(idx_hbm, ewb_hbm, zero_hbm, out_hbm,
             idx0, idx1, ewb0, ewb1, rows0, rows1, acc,
             si0, si1, se0, se1, ss0):
        c = lax.axis_index("c")
        s = lax.axis_index("s")
        wid = c * _NS + s
        row0 = s * rpt
        pltpu.sync_copy(zero_hbm.at[pl.ds(row0, rpt)], acc.at[pl.ds(row0, rpt)])
        plsc.subcore_barrier()

        pltpu.sync_copy(idx_hbm.at[wid, 0], idx0)
        pltpu.sync_copy(ewb_hbm.at[wid, 0], ewb0)

        def pair(i, carry):
            a = 2 * i
            b = a + 1
            pltpu.async_copy(idx_hbm.at[wid, b], idx1, si1)
            pltpu.async_copy(ewb_hbm.at[wid, b], ewb1, se1)
            _fill_rows(rows0, ewb0)
            pltpu.make_async_copy(idx_hbm.at[wid, b], idx1, si1).wait()
            pltpu.make_async_copy(ewb_hbm.at[wid, b], ewb1, se1).wait()
            pltpu.async_copy(rows0, acc.at[idx0.at[1]], ss0, add=True)
            _fill_rows(rows1, ewb1)
            pltpu.make_async_copy(rows0, acc.at[idx0.at[1]], ss0).wait()
            pltpu.async_copy(idx_hbm.at[wid, a + 2], idx0, si0)
            pltpu.async_copy(ewb_hbm.at[wid, a + 2], ewb0, se0)
            pltpu.sync_copy(rows1, acc.at[idx1.at[1]], add=True)
            pltpu.make_async_copy(idx_hbm.at[wid, a + 2], idx0, si0).wait()
            pltpu.make_async_copy(ewb_hbm.at[wid, a + 2], ewb0, se0).wait()
            return carry

        trip = lax.select(c == 0, ne0 // 2, ne1 // 2)
        lax.fori_loop(0, trip, pair, 0)
        plsc.subcore_barrier()
        pltpu.sync_copy(acc.at[pl.ds(row0, rpt)], out_hbm.at[c, pl.ds(row0, rpt)])

    return pl.kernel(
        body,
        out_type=jax.ShapeDtypeStruct((_NC, n_pad, 128), jnp.float32),
        mesh=_sc_mesh(),
        scratch_types=[
            pltpu.VMEM((2, _K), jnp.int32),
            pltpu.VMEM((2, _K), jnp.int32),
            pltpu.VMEM((_K, _DW), jnp.float32),
            pltpu.VMEM((_K, _DW), jnp.float32),
            pltpu.VMEM((_K, 128), jnp.float32),
            pltpu.VMEM((_K, 128), jnp.float32),
            pltpu.VMEM_SHARED((n_pad, 128), jnp.float32),
            pltpu.SemaphoreType.DMA,
            pltpu.SemaphoreType.DMA,
            pltpu.SemaphoreType.DMA,
            pltpu.SemaphoreType.DMA,
            pltpu.SemaphoreType.DMA,
        ],
    )


def _agg_kernel(d, n_pad, ne0, ne1, rpt):
    def body(h_hbm, idx_hbm, ewb_hbm, zero_hbm, out_hbm,
             idx0, idx1, ewb0, ewb1, rows0, rows1, acc,
             sg0, sg1, si0, si1, se0, se1, ss0):
        c = lax.axis_index("c")
        s = lax.axis_index("s")
        wid = c * _NS + s
        row0 = s * rpt
        pltpu.sync_copy(zero_hbm.at[pl.ds(row0, rpt)], acc.at[pl.ds(row0, rpt)])
        plsc.subcore_barrier()

        pltpu.sync_copy(idx_hbm.at[wid, 0], idx0)
        pltpu.sync_copy(ewb_hbm.at[wid, 0], ewb0)
        pltpu.async_copy(h_hbm.at[idx0.at[0]], rows0, sg0)

        def pair(i, carry):
            a = 2 * i
            b = a + 1
            pltpu.async_copy(idx_hbm.at[wid, b], idx1, si1)
            pltpu.async_copy(ewb_hbm.at[wid, b], ewb1, se1)
            pltpu.make_async_copy(h_hbm.at[idx0.at[0]], rows0, sg0).wait()
            _scale_rows(rows0, ewb0, d)
            pltpu.make_async_copy(idx_hbm.at[wid, b], idx1, si1).wait()
            pltpu.make_async_copy(ewb_hbm.at[wid, b], ewb1, se1).wait()
            pltpu.async_copy(h_hbm.at[idx1.at[0]], rows1, sg1)
            pltpu.async_copy(rows0, acc.at[idx0.at[1]], ss0, add=True)
            pltpu.make_async_copy(h_hbm.at[idx1.at[0]], rows1, sg1).wait()
            _scale_rows(rows1, ewb1, d)
            pltpu.make_async_copy(rows0, acc.at[idx0.at[1]], ss0).wait()
            pltpu.async_copy(idx_hbm.at[wid, a + 2], idx0, si0)
            pltpu.async_copy(ewb_hbm.at[wid, a + 2], ewb0, se0)
            pltpu.sync_copy(rows1, acc.at[idx1.at[1]], add=True)
            pltpu.make_async_copy(idx_hbm.at[wid, a + 2], idx0, si0).wait()
            pltpu.make_async_copy(ewb_hbm.at[wid, a + 2], ewb0, se0).wait()
            pltpu.async_copy(h_hbm.at[idx0.at[0]], rows0, sg0)
            return carry

        trip = lax.select(c == 0, ne0 // 2, ne1 // 2)
        lax.fori_loop(0, trip, pair, 0)
        pltpu.make_async_copy(h_hbm.at[idx0.at[0]], rows0, sg0).wait()
        plsc.subcore_barrier()
        pltpu.sync_copy(acc.at[pl.ds(row0, rpt)], out_hbm.at[c, pl.ds(row0, rpt)])

    return pl.kernel(
        body,
        out_type=jax.ShapeDtypeStruct((_NC, n_pad, d), jnp.float32),
        mesh=_sc_mesh(),
        scratch_types=[
            pltpu.VMEM((2, _K), jnp.int32),
            pltpu.VMEM((2, _K), jnp.int32),
            pltpu.VMEM((_K, _DW), jnp.float32),
            pltpu.VMEM((_K, _DW), jnp.float32),
            pltpu.VMEM((_K, d), jnp.float32),
            pltpu.VMEM((_K, d), jnp.float32),
            pltpu.VMEM_SHARED((n_pad, d), jnp.float32),
            pltpu.SemaphoreType.DMA,
            pltpu.SemaphoreType.DMA,
            pltpu.SemaphoreType.DMA,
            pltpu.SemaphoreType.DMA,
            pltpu.SemaphoreType.DMA,
            pltpu.SemaphoreType.DMA,
            pltpu.SemaphoreType.DMA,
        ],
    )


def _tc_mm(n, d):
    def body(x_ref, w_ref, out_ref):
        out_ref[...] = jnp.dot(x_ref[...], w_ref[...],
                               preferred_element_type=jnp.float32)

    return pl.pallas_call(
        body,
        out_shape=[jax.ShapeDtypeStruct((n, d), jnp.float32)],
    )


def _tc_prep(n, d):
    def body(m_ref, dp_ref, dis_ref, hp_ref):
        deg = dp_ref[0, :n, 0:1] + dp_ref[1, :n, 0:1] + 1.0
        dis = lax.rsqrt(deg)
        dis_ref[...] = dis
        hp_ref[...] = m_ref[...] * dis

    return pl.pallas_call(
        body,
        out_shape=[
            jax.ShapeDtypeStruct((n, 1), jnp.float32),
            jax.ShapeDtypeStruct((n, d), jnp.float32),
        ],
    )


def _tc_mid(n, d):
    def body(ap_ref, hp_ref, dis_ref, b_ref, w_ref, out_ref):
        dis = dis_ref[...]
        pre = (ap_ref[0, :n, :] + ap_ref[1, :n, :] + hp_ref[...]) * dis + b_ref[...]
        h = jnp.maximum(pre, 0.0)
        out_ref[...] = jnp.dot(h, w_ref[...], preferred_element_type=jnp.float32) * dis

    return pl.pallas_call(
        body,
        out_shape=[jax.ShapeDtypeStruct((n, d), jnp.float32)],
    )


def _tc_head(n, d, nb, dout):
    def body(ap_ref, hp_ref, dis_ref, b_ref, batch_ref, bs_ref, u_ref,
             wha_ref, whb_ref, bh1_ref, wh2_ref, bh2_ref, out_ref):
        dis = dis_ref[...]
        h2 = jnp.maximum(
            (ap_ref[0, :n, :] + ap_ref[1, :n, :] + hp_ref[...]) * dis + b_ref[...],
            0.0)
        seg = lax.broadcasted_iota(jnp.int32, (nb, n), 0)
        onehot = (seg == batch_ref[...]).astype(jnp.float32)
        sums = jnp.dot(onehot, h2, preferred_element_type=jnp.float32)
        cnts = jnp.sum(onehot, axis=1, keepdims=True)
        gem = sums / jnp.maximum(cnts, 1.0)
        valid = lax.broadcasted_iota(jnp.int32, (nb, 1), 0) < bs_ref[0]
        gem = jnp.where(valid, gem, 0.0)
        z = jnp.maximum(
            jnp.dot(gem, wha_ref[...], preferred_element_type=jnp.float32)
            + jnp.dot(u_ref[...], whb_ref[...], preferred_element_type=jnp.float32)
            + bh1_ref[...], 0.0)
        out_ref[...] = (jnp.dot(z, wh2_ref[...], preferred_element_type=jnp.float32)
                        + bh2_ref[...])

    nargs = 12
    specs = [pl.BlockSpec(memory_space=pltpu.VMEM) for _ in range(nargs)]
    specs[5] = pl.BlockSpec(memory_space=pltpu.SMEM)
    return pl.pallas_call(
        body,
        in_specs=specs,
        out_specs=[pl.BlockSpec(memory_space=pltpu.VMEM)],
        out_shape=[jax.ShapeDtypeStruct((nb, dout), jnp.float32)],
    )


def kernel(x, edge_index, u, edge_weight, batch, batch_size,
           W1, b1, W2, b2, Wh1, bh1, Wh2, bh2):
    n, _ = x.shape
    d = W1.shape[1]
    e = edge_weight.shape[0]
    nb, dg = u.shape
    dout = Wh2.shape[1]

    grp0 = _NS * _K
    ne0 = -(-int(e * _FRAC0) // grp0)
    ne0 = ne0 + (ne0 % 2)
    e0 = min(e, grp0 * ne0)
    ne1 = -(-(e - e0) // grp0)
    ne1 = ne1 + (ne1 % 2)
    nm = max(ne0, ne1) + 2
    rpt = (-(-n // _NS) + 7) // 8 * 8
    n_pad = rpt * _NS

    def split(arr, dtype):
        a0 = arr[:e0].reshape(_NS, ne0, _K)
        a0 = jnp.pad(a0, ((0, 0), (0, nm - ne0), (0, 0)))
        a1 = jnp.pad(arr[e0:], (0, grp0 * ne1 - (e - e0)))
        a1 = jnp.pad(a1.reshape(_NS, ne1, _K), ((0, 0), (0, nm - ne1), (0, 0)))
        return jnp.concatenate([a0, a1], axis=0).astype(dtype)

    srcp = split(edge_index[0], jnp.int32)[:, :, None, :]
    dstp = split(edge_index[1], jnp.int32)[:, :, None, :]
    idxp = jnp.concatenate([srcp, dstp], axis=2)
    ewb = jnp.broadcast_to(
        split(edge_weight, jnp.float32)[:, :, :, None],
        (_NW, nm, _K, _DW))
    zero_w = jnp.zeros((n_pad, d), jnp.float32)

    (m1,) = _tc_mm(n, d)(x, W1)
    degp = _deg_kernel(n_pad, ne0, ne1, rpt)(idxp, ewb, zero_w)
    dis, h1p = _tc_prep(n, d)(m1, degp)

    agg = _agg_kernel(d, n_pad, ne0, ne1, rpt)
    a1 = agg(h1p, idxp, ewb, zero_w)
    (h2p,) = _tc_mid(n, d)(a1, h1p, dis, b1.reshape(1, d), W2)
    a2 = agg(h2p, idxp, ewb, zero_w)

    (out,) = _tc_head(n, d, nb, dout)(
        a2, h2p, dis, b2.reshape(1, d),
        batch.reshape(1, n).astype(jnp.int32),
        jnp.asarray(batch_size, jnp.int32).reshape(1),
        u, Wh1[:d], Wh1[d:], bh1.reshape(1, d), Wh2, bh2.reshape(1, dout))
    return out


# speedup vs baseline: 1.0858x; 1.0069x over previous
import jax
import jax.numpy as jnp
from jax import lax
from jax.experimental import pallas as pl
from jax.experimental.pallas import tpu as pltpu
from jax.experimental.pallas import tpu_sc as plsc

_NC = 2
_NS = 16
_NW = _NC * _NS
_L = 16
_K = 80
_DW = 16
_FRAC0 = 0.50


def _sc_mesh():
    return plsc.VectorSubcoreMesh(core_axis_name="c", subcore_axis_name="s",
                                  num_cores=_NC, num_subcores=_NS)


def _scale_rows(rows, ewb_v, d):
    def grp(g, carry):
        for l in range(_L):
            e = g * _L + l
            w = ewb_v[e, :]
            for j in range(d // _L):
                sl = pl.ds(j * _L, _L)
                rows[e, sl] = rows[e, sl] * w
        return carry

    lax.fori_loop(0, _K // _L, grp, 0)


def _fill_rows(rows, ewb_v):
    def grp(g, carry):
        for l in range(_L):
            e = g * _L + l
            w = ewb_v[e, :]
            for j in range(128 // _L):
                rows[e, pl.ds(j * _L, _L)] = w
        return carry

    lax.fori_loop(0, _K // _L, grp, 0)


def _deg_kernel(n_pad, ne0, ne1, rpt):
    def body(idx_hbm, ewb_hbm, zero_hbm, out_hbm,
             idx0, idx1, ewb0, ewb1, rows0, rows1, acc,
             si0, si1, se0, se1, ss0):
        c = lax.axis_index("c")
        s = lax.axis_index("s")
        wid = c * _NS + s
        row0 = s * rpt
        pltpu.sync_copy(zero_hbm.at[pl.ds(row0, rpt)], acc.at[pl.ds(row0, rpt)])
        plsc.subcore_barrier()

        pltpu.sync_copy(idx_hbm.at[wid, 0], idx0)
        pltpu.sync_copy(ewb_hbm.at[wid, 0], ewb0)

        def pair(i, carry):
            a = 2 * i
            b = a + 1
            pltpu.async_copy(idx_hbm.at[wid, b], idx1, si1)
            pltpu.async_copy(ewb_hbm.at[wid, b], ewb1, se1)
            _fill_rows(rows0, ewb0)
            pltpu.make_async_copy(idx_hbm.at[wid, b], idx1, si1).wait()
            pltpu.make_async_copy(ewb_hbm.at[wid, b], ewb1, se1).wait()
            pltpu.async_copy(rows0, acc.at[idx0.at[1]], ss0, add=True)
            _fill_rows(rows1, ewb1)
            pltpu.make_async_copy(rows0, acc.at[idx0.at[1]], ss0).wait()
            pltpu.async_copy(idx_hbm.at[wid, a + 2], idx0, si0)
            pltpu.async_copy(ewb_hbm.at[wid, a + 2], ewb0, se0)
            pltpu.sync_copy(rows1, acc.at[idx1.at[1]], add=True)
            pltpu.make_async_copy(idx_hbm.at[wid, a + 2], idx0, si0).wait()
            pltpu.make_async_copy(ewb_hbm.at[wid, a + 2], ewb0, se0).wait()
            return carry

        trip = lax.select(c == 0, ne0 // 2, ne1 // 2)
        lax.fori_loop(0, trip, pair, 0)
        plsc.subcore_barrier()
        pltpu.sync_copy(acc.at[pl.ds(row0, rpt)], out_hbm.at[c, pl.ds(row0, rpt)])

    return pl.kernel(
        body,
        out_type=jax.ShapeDtypeStruct((_NC, n_pad, 128), jnp.float32),
        mesh=_sc_mesh(),
        scratch_types=[
            pltpu.VMEM((2, _K), jnp.int32),
            pltpu.VMEM((2, _K), jnp.int32),
            pltpu.VMEM((_K, _DW), jnp.float32),
            pltpu.VMEM((_K, _DW), jnp.float32),
            pltpu.VMEM((_K, 128), jnp.float32),
            pltpu.VMEM((_K, 128), jnp.float32),
            pltpu.VMEM_SHARED((n_pad, 128), jnp.float32),
            pltpu.SemaphoreType.DMA,
            pltpu.SemaphoreType.DMA,
            pltpu.SemaphoreType.DMA,
            pltpu.SemaphoreType.DMA,
            pltpu.SemaphoreType.DMA,
        ],
    )


def _agg_kernel(d, n_pad, ne0, ne1, rpt):
    def body(h_hbm, idx_hbm, ewb_hbm, zero_hbm, out_hbm,
             idx0, idx1, ewb0, ewb1, rows0, rows1, acc,
             sg0, sg1, si0, si1, se0, se1, ss0):
        c = lax.axis_index("c")
        s = lax.axis_index("s")
        wid = c * _NS + s
        row0 = s * rpt
        pltpu.sync_copy(zero_hbm.at[pl.ds(row0, rpt)], acc.at[pl.ds(row0, rpt)])
        plsc.subcore_barrier()

        pltpu.sync_copy(idx_hbm.at[wid, 0], idx0)
        pltpu.sync_copy(ewb_hbm.at[wid, 0], ewb0)
        pltpu.async_copy(h_hbm.at[idx0.at[0]], rows0, sg0)

        def pair(i, carry):
            a = 2 * i
            b = a + 1
            pltpu.async_copy(idx_hbm.at[wid, b], idx1, si1)
            pltpu.async_copy(ewb_hbm.at[wid, b], ewb1, se1)
            pltpu.make_async_copy(h_hbm.at[idx0.at[0]], rows0, sg0).wait()
            _scale_rows(rows0, ewb0, d)
            pltpu.make_async_copy(idx_hbm.at[wid, b], idx1, si1).wait()
            pltpu.make_async_copy(ewb_hbm.at[wid, b], ewb1, se1).wait()
            pltpu.async_copy(h_hbm.at[idx1.at[0]], rows1, sg1)
            pltpu.async_copy(rows0, acc.at[idx0.at[1]], ss0, add=True)
            pltpu.make_async_copy(h_hbm.at[idx1.at[0]], rows1, sg1).wait()
            _scale_rows(rows1, ewb1, d)
            pltpu.make_async_copy(rows0, acc.at[idx0.at[1]], ss0).wait()
            pltpu.async_copy(idx_hbm.at[wid, a + 2], idx0, si0)
            pltpu.async_copy(ewb_hbm.at[wid, a + 2], ewb0, se0)
            pltpu.sync_copy(rows1, acc.at[idx1.at[1]], add=True)
            pltpu.make_async_copy(idx_hbm.at[wid, a + 2], idx0, si0).wait()
            pltpu.make_async_copy(ewb_hbm.at[wid, a + 2], ewb0, se0).wait()
            pltpu.async_copy(h_hbm.at[idx0.at[0]], rows0, sg0)
            return carry

        trip = lax.select(c == 0, ne0 // 2, ne1 // 2)
        lax.fori_loop(0, trip, pair, 0)
        pltpu.make_async_copy(h_hbm.at[idx0.at[0]], rows0, sg0).wait()
        plsc.subcore_barrier()
        pltpu.sync_copy(acc.at[pl.ds(row0, rpt)], out_hbm.at[c, pl.ds(row0, rpt)])

    return pl.kernel(
        body,
        out_type=jax.ShapeDtypeStruct((_NC, n_pad, d), jnp.float32),
        mesh=_sc_mesh(),
        scratch_types=[
            pltpu.VMEM((2, _K), jnp.int32),
            pltpu.VMEM((2, _K), jnp.int32),
            pltpu.VMEM((_K, _DW), jnp.float32),
            pltpu.VMEM((_K, _DW), jnp.float32),
            pltpu.VMEM((_K, d), jnp.float32),
            pltpu.VMEM((_K, d), jnp.float32),
            pltpu.VMEM_SHARED((n_pad, d), jnp.float32),
            pltpu.SemaphoreType.DMA,
            pltpu.SemaphoreType.DMA,
            pltpu.SemaphoreType.DMA,
            pltpu.SemaphoreType.DMA,
            pltpu.SemaphoreType.DMA,
            pltpu.SemaphoreType.DMA,
            pltpu.SemaphoreType.DMA,
        ],
    )


def _tc_mm(n, d):
    def body(x_ref, w_ref, out_ref):
        out_ref[...] = jnp.dot(x_ref[...], w_ref[...],
                               preferred_element_type=jnp.float32)

    return pl.pallas_call(
        body,
        out_shape=[jax.ShapeDtypeStruct((n, d), jnp.float32)],
    )


def _tc_prep(n, d):
    def body(m_ref, dp_ref, dis_ref, hp_ref):
        deg = dp_ref[0, :n, 0:1] + dp_ref[1, :n, 0:1] + 1.0
        dis = lax.rsqrt(deg)
        dis_ref[...] = dis
        hp_ref[...] = m_ref[...] * dis

    return pl.pallas_call(
        body,
        out_shape=[
            jax.ShapeDtypeStruct((n, 1), jnp.float32),
            jax.ShapeDtypeStruct((n, d), jnp.float32),
        ],
    )


def _tc_mid(n, d):
    def body(ap_ref, hp_ref, dis_ref, b_ref, w_ref, out_ref):
        dis = dis_ref[...]
        pre = (ap_ref[0, :n, :] + ap_ref[1, :n, :] + hp_ref[...]) * dis + b_ref[...]
        h = jnp.maximum(pre, 0.0)
        out_ref[...] = jnp.dot(h, w_ref[...], preferred_element_type=jnp.float32) * dis

    return pl.pallas_call(
        body,
        out_shape=[jax.ShapeDtypeStruct((n, d), jnp.float32)],
    )


def _tc_head(n, d, nb, dout):
    def body(ap_ref, hp_ref, dis_ref, b_ref, batch_ref, bs_ref, u_ref,
             wha_ref, whb_ref, bh1_ref, wh2_ref, bh2_ref, out_ref):
        dis = dis_ref[...]
        h2 = jnp.maximum(
            (ap_ref[0, :n, :] + ap_ref[1, :n, :] + hp_ref[...]) * dis + b_ref[...],
            0.0)
        seg = lax.broadcasted_iota(jnp.int32, (nb, n), 0)
        onehot = (seg == batch_ref[...]).astype(jnp.float32)
        sums = jnp.dot(onehot, h2, preferred_element_type=jnp.float32)
        cnts = jnp.sum(onehot, axis=1, keepdims=True)
        gem = sums / jnp.maximum(cnts, 1.0)
        valid = lax.broadcasted_iota(jnp.int32, (nb, 1), 0) < bs_ref[0]
        gem = jnp.where(valid, gem, 0.0)
        z = jnp.maximum(
            jnp.dot(gem, wha_ref[...], preferred_element_type=jnp.float32)
            + jnp.dot(u_ref[...], whb_ref[...], preferred_element_type=jnp.float32)
            + bh1_ref[...], 0.0)
        out_ref[...] = (jnp.dot(z, wh2_ref[...], preferred_element_type=jnp.float32)
                        + bh2_ref[...])

    nargs = 12
    specs = [pl.BlockSpec(memory_space=pltpu.VMEM) for _ in range(nargs)]
    specs[5] = pl.BlockSpec(memory_space=pltpu.SMEM)
    return pl.pallas_call(
        body,
        in_specs=specs,
        out_specs=[pl.BlockSpec(memory_space=pltpu.VMEM)],
        out_shape=[jax.ShapeDtypeStruct((nb, dout), jnp.float32)],
    )


def kernel(x, edge_index, u, edge_weight, batch, batch_size,
           W1, b1, W2, b2, Wh1, bh1, Wh2, bh2):
    n, _ = x.shape
    d = W1.shape[1]
    e = edge_weight.shape[0]
    nb, dg = u.shape
    dout = Wh2.shape[1]

    grp0 = _NS * _K
    ne0 = -(-int(e * _FRAC0) // grp0)
    ne0 = ne0 + (ne0 % 2)
    e0 = min(e, grp0 * ne0)
    ne1 = -(-(e - e0) // grp0)
    ne1 = ne1 + (ne1 % 2)
    nm = max(ne0, ne1) + 2
    rpt = (-(-n // _NS) + 7) // 8 * 8
    n_pad = rpt * _NS

    def split(arr, dtype):
        a0 = arr[:e0].reshape(_NS, ne0, _K)
        a0 = jnp.pad(a0, ((0, 0), (0, nm - ne0), (0, 0)))
        a1 = jnp.pad(arr[e0:], (0, grp0 * ne1 - (e - e0)))
        a1 = jnp.pad(a1.reshape(_NS, ne1, _K), ((0, 0), (0, nm - ne1), (0, 0)))
        return jnp.concatenate([a0, a1], axis=0).astype(dtype)

    srcp = split(edge_index[0], jnp.int32)[:, :, None, :]
    dstp = split(edge_index[1], jnp.int32)[:, :, None, :]
    idxp = jnp.concatenate([srcp, dstp], axis=2)
    ewb = jnp.broadcast_to(
        split(edge_weight, jnp.float32)[:, :, :, None],
        (_NW, nm, _K, _DW))
    zero_w = jnp.zeros((n_pad, d), jnp.float32)

    (m1,) = _tc_mm(n, d)(x, W1)
    degp = _deg_kernel(n_pad, ne0, ne1, rpt)(idxp, ewb, zero_w)
    dis, h1p = _tc_prep(n, d)(m1, degp)

    agg = _agg_kernel(d, n_pad, ne0, ne1, rpt)
    a1 = agg(h1p, idxp, ewb, zero_w)
    (h2p,) = _tc_mid(n, d)(a1, h1p, dis, b1.reshape(1, d), W2)
    a2 = agg(h2p, idxp, ewb, zero_w)

    (out,) = _tc_head(n, d, nb, dout)(
        a2, h2p, dis, b2.reshape(1, d),
        batch.reshape(1, n).astype(jnp.int32),
        jnp.asarray(batch_size, jnp.int32).reshape(1),
        u, Wh1[:d], Wh1[d:], bh1.reshape(1, d), Wh2, bh2.reshape(1, dout))
    return out


# speedup vs baseline: 1.0902x; 1.0040x over previous
import jax
import jax.numpy as jnp
from jax import lax
from jax.experimental import pallas as pl
from jax.experimental.pallas import tpu as pltpu
from jax.experimental.pallas import tpu_sc as plsc

_NC = 2
_NS = 16
_NW = _NC * _NS
_L = 16
_K = 80
_DW = 16
_FRAC0 = 0.50


def _sc_mesh():
    return plsc.VectorSubcoreMesh(core_axis_name="c", subcore_axis_name="s",
                                  num_cores=_NC, num_subcores=_NS)


def _scale_rows(rows, ewb_v, d):
    def grp(g, carry):
        for l in range(_L):
            e = g * _L + l
            w = ewb_v[e, :]
            for j in range(d // _L):
                sl = pl.ds(j * _L, _L)
                rows[e, sl] = rows[e, sl] * w
        return carry

    lax.fori_loop(0, _K // _L, grp, 0)


def _zero_stripe(rows, acc, row0, nrep):
    zv = jnp.zeros((_L,), jnp.float32)

    def grp(g, carry):
        for l in range(_L):
            e = g * _L + l
            for j in range(128 // _L):
                rows[e, pl.ds(j * _L, _L)] = zv
        return carry

    lax.fori_loop(0, _K // _L, grp, 0)
    for i in range(nrep):
        pltpu.sync_copy(rows, acc.at[pl.ds(row0 + i * _K, _K)])


def _fill_rows(rows, ewb_v):
    def grp(g, carry):
        for l in range(_L):
            e = g * _L + l
            w = ewb_v[e, :]
            for j in range(128 // _L):
                rows[e, pl.ds(j * _L, _L)] = w
        return carry

    lax.fori_loop(0, _K // _L, grp, 0)


def _deg_kernel(n_pad, ne0, ne1, rpt):
    def body(idx_hbm, ewb_hbm, out_hbm,
             idx0, idx1, ewb0, ewb1, rows0, rows1, acc,
             si0, si1, se0, se1, ss0):
        c = lax.axis_index("c")
        s = lax.axis_index("s")
        wid = c * _NS + s
        row0 = s * rpt
        _zero_stripe(rows0, acc, row0, rpt // _K)
        plsc.subcore_barrier()

        pltpu.sync_copy(idx_hbm.at[wid, 0], idx0)
        pltpu.sync_copy(ewb_hbm.at[wid, 0], ewb0)

        def pair(i, carry):
            a = 2 * i
            b = a + 1
            pltpu.async_copy(idx_hbm.at[wid, b], idx1, si1)
            pltpu.async_copy(ewb_hbm.at[wid, b], ewb1, se1)
            _fill_rows(rows0, ewb0)
            pltpu.make_async_copy(idx_hbm.at[wid, b], idx1, si1).wait()
            pltpu.make_async_copy(ewb_hbm.at[wid, b], ewb1, se1).wait()
            pltpu.async_copy(rows0, acc.at[idx0.at[1]], ss0, add=True)
            _fill_rows(rows1, ewb1)
            pltpu.make_async_copy(rows0, acc.at[idx0.at[1]], ss0).wait()
            pltpu.async_copy(idx_hbm.at[wid, a + 2], idx0, si0)
            pltpu.async_copy(ewb_hbm.at[wid, a + 2], ewb0, se0)
            pltpu.sync_copy(rows1, acc.at[idx1.at[1]], add=True)
            pltpu.make_async_copy(idx_hbm.at[wid, a + 2], idx0, si0).wait()
            pltpu.make_async_copy(ewb_hbm.at[wid, a + 2], ewb0, se0).wait()
            return carry

        trip = lax.select(c == 0, ne0 // 2, ne1 // 2)
        lax.fori_loop(0, trip, pair, 0)
        plsc.subcore_barrier()
        pltpu.sync_copy(acc.at[pl.ds(row0, rpt)], out_hbm.at[c, pl.ds(row0, rpt)])

    return pl.kernel(
        body,
        out_type=jax.ShapeDtypeStruct((_NC, n_pad, 128), jnp.float32),
        mesh=_sc_mesh(),
        scratch_types=[
            pltpu.VMEM((2, _K), jnp.int32),
            pltpu.VMEM((2, _K), jnp.int32),
            pltpu.VMEM((_K, _DW), jnp.float32),
            pltpu.VMEM((_K, _DW), jnp.float32),
            pltpu.VMEM((_K, 128), jnp.float32),
            pltpu.VMEM((_K, 128), jnp.float32),
            pltpu.VMEM_SHARED((n_pad, 128), jnp.float32),
            pltpu.SemaphoreType.DMA,
            pltpu.SemaphoreType.DMA,
            pltpu.SemaphoreType.DMA,
            pltpu.SemaphoreType.DMA,
            pltpu.SemaphoreType.DMA,
        ],
    )


def _agg_kernel(d, n_pad, ne0, ne1, rpt):
    def body(h_hbm, idx_hbm, ewb_hbm, out_hbm,
             idx0, idx1, ewb0, ewb1, rows0, rows1, acc,
             sg0, sg1, si0, si1, se0, se1, ss0):
        c = lax.axis_index("c")
        s = lax.axis_index("s")
        wid = c * _NS + s
        row0 = s * rpt
        _zero_stripe(rows0, acc, row0, rpt // _K)
        plsc.subcore_barrier()

        pltpu.sync_copy(idx_hbm.at[wid, 0], idx0)
        pltpu.sync_copy(ewb_hbm.at[wid, 0], ewb0)
        pltpu.async_copy(h_hbm.at[idx0.at[0]], rows0, sg0)

        def pair(i, carry):
            a = 2 * i
            b = a + 1
            pltpu.async_copy(idx_hbm.at[wid, b], idx1, si1)
            pltpu.async_copy(ewb_hbm.at[wid, b], ewb1, se1)
            pltpu.make_async_copy(h_hbm.at[idx0.at[0]], rows0, sg0).wait()
            _scale_rows(rows0, ewb0, d)
            pltpu.make_async_copy(idx_hbm.at[wid, b], idx1, si1).wait()
            pltpu.make_async_copy(ewb_hbm.at[wid, b], ewb1, se1).wait()
            pltpu.async_copy(h_hbm.at[idx1.at[0]], rows1, sg1)
            pltpu.async_copy(rows0, acc.at[idx0.at[1]], ss0, add=True)
            pltpu.make_async_copy(h_hbm.at[idx1.at[0]], rows1, sg1).wait()
            _scale_rows(rows1, ewb1, d)
            pltpu.make_async_copy(rows0, acc.at[idx0.at[1]], ss0).wait()
            pltpu.async_copy(idx_hbm.at[wid, a + 2], idx0, si0)
            pltpu.async_copy(ewb_hbm.at[wid, a + 2], ewb0, se0)
            pltpu.sync_copy(rows1, acc.at[idx1.at[1]], add=True)
            pltpu.make_async_copy(idx_hbm.at[wid, a + 2], idx0, si0).wait()
            pltpu.make_async_copy(ewb_hbm.at[wid, a + 2], ewb0, se0).wait()
            pltpu.async_copy(h_hbm.at[idx0.at[0]], rows0, sg0)
            return carry

        trip = lax.select(c == 0, ne0 // 2, ne1 // 2)
        lax.fori_loop(0, trip, pair, 0)
        pltpu.make_async_copy(h_hbm.at[idx0.at[0]], rows0, sg0).wait()
        plsc.subcore_barrier()
        pltpu.sync_copy(acc.at[pl.ds(row0, rpt)], out_hbm.at[c, pl.ds(row0, rpt)])

    return pl.kernel(
        body,
        out_type=jax.ShapeDtypeStruct((_NC, n_pad, d), jnp.float32),
        mesh=_sc_mesh(),
        scratch_types=[
            pltpu.VMEM((2, _K), jnp.int32),
            pltpu.VMEM((2, _K), jnp.int32),
            pltpu.VMEM((_K, _DW), jnp.float32),
            pltpu.VMEM((_K, _DW), jnp.float32),
            pltpu.VMEM((_K, d), jnp.float32),
            pltpu.VMEM((_K, d), jnp.float32),
            pltpu.VMEM_SHARED((n_pad, d), jnp.float32),
            pltpu.SemaphoreType.DMA,
            pltpu.SemaphoreType.DMA,
            pltpu.SemaphoreType.DMA,
            pltpu.SemaphoreType.DMA,
            pltpu.SemaphoreType.DMA,
            pltpu.SemaphoreType.DMA,
            pltpu.SemaphoreType.DMA,
        ],
    )


def _tc_mm(n, d):
    def body(x_ref, w_ref, out_ref):
        out_ref[...] = jnp.dot(x_ref[...], w_ref[...],
                               preferred_element_type=jnp.float32)

    return pl.pallas_call(
        body,
        out_shape=[jax.ShapeDtypeStruct((n, d), jnp.float32)],
    )


def _tc_prep(n, d):
    def body(m_ref, dp_ref, dis_ref, hp_ref):
        deg = dp_ref[0, :n, 0:1] + dp_ref[1, :n, 0:1] + 1.0
        dis = lax.rsqrt(deg)
        dis_ref[...] = dis
        hp_ref[...] = m_ref[...] * dis

    return pl.pallas_call(
        body,
        out_shape=[
            jax.ShapeDtypeStruct((n, 1), jnp.float32),
            jax.ShapeDtypeStruct((n, d), jnp.float32),
        ],
    )


def _tc_mid(n, d):
    def body(ap_ref, hp_ref, dis_ref, b_ref, w_ref, out_ref):
        dis = dis_ref[...]
        pre = (ap_ref[0, :n, :] + ap_ref[1, :n, :] + hp_ref[...]) * dis + b_ref[...]
        h = jnp.maximum(pre, 0.0)
        out_ref[...] = jnp.dot(h, w_ref[...], preferred_element_type=jnp.float32) * dis

    return pl.pallas_call(
        body,
        out_shape=[jax.ShapeDtypeStruct((n, d), jnp.float32)],
    )


def _tc_head(n, d, nb, dout):
    def body(ap_ref, hp_ref, dis_ref, b_ref, batch_ref, bs_ref, u_ref,
             wha_ref, whb_ref, bh1_ref, wh2_ref, bh2_ref, out_ref):
        dis = dis_ref[...]
        h2 = jnp.maximum(
            (ap_ref[0, :n, :] + ap_ref[1, :n, :] + hp_ref[...]) * dis + b_ref[...],
            0.0)
        seg = lax.broadcasted_iota(jnp.int32, (nb, n), 0)
        onehot = (seg == batch_ref[...]).astype(jnp.float32)
        sums = jnp.dot(onehot, h2, preferred_element_type=jnp.float32)
        cnts = jnp.sum(onehot, axis=1, keepdims=True)
        gem = sums / jnp.maximum(cnts, 1.0)
        valid = lax.broadcasted_iota(jnp.int32, (nb, 1), 0) < bs_ref[0]
        gem = jnp.where(valid, gem, 0.0)
        z = jnp.maximum(
            jnp.dot(gem, wha_ref[...], preferred_element_type=jnp.float32)
            + jnp.dot(u_ref[...], whb_ref[...], preferred_element_type=jnp.float32)
            + bh1_ref[...], 0.0)
        out_ref[...] = (jnp.dot(z, wh2_ref[...], preferred_element_type=jnp.float32)
                        + bh2_ref[...])

    nargs = 12
    specs = [pl.BlockSpec(memory_space=pltpu.VMEM) for _ in range(nargs)]
    specs[5] = pl.BlockSpec(memory_space=pltpu.SMEM)
    return pl.pallas_call(
        body,
        in_specs=specs,
        out_specs=[pl.BlockSpec(memory_space=pltpu.VMEM)],
        out_shape=[jax.ShapeDtypeStruct((nb, dout), jnp.float32)],
    )


def kernel(x, edge_index, u, edge_weight, batch, batch_size,
           W1, b1, W2, b2, Wh1, bh1, Wh2, bh2):
    n, _ = x.shape
    d = W1.shape[1]
    e = edge_weight.shape[0]
    nb, dg = u.shape
    dout = Wh2.shape[1]

    grp0 = _NS * _K
    ne0 = -(-int(e * _FRAC0) // grp0)
    ne0 = ne0 + (ne0 % 2)
    e0 = min(e, grp0 * ne0)
    ne1 = -(-(e - e0) // grp0)
    ne1 = ne1 + (ne1 % 2)
    nm = max(ne0, ne1) + 2
    rpt = -(-(-(-n // _NS)) // _K) * _K
    n_pad = rpt * _NS

    def split(arr, dtype):
        a0 = arr[:e0].reshape(_NS, ne0, _K)
        a0 = jnp.pad(a0, ((0, 0), (0, nm - ne0), (0, 0)))
        a1 = jnp.pad(arr[e0:], (0, grp0 * ne1 - (e - e0)))
        a1 = jnp.pad(a1.reshape(_NS, ne1, _K), ((0, 0), (0, nm - ne1), (0, 0)))
        return jnp.concatenate([a0, a1], axis=0).astype(dtype)

    srcp = split(edge_index[0], jnp.int32)[:, :, None, :]
    dstp = split(edge_index[1], jnp.int32)[:, :, None, :]
    idxp = jnp.concatenate([srcp, dstp], axis=2)
    ewb = jnp.broadcast_to(
        split(edge_weight, jnp.float32)[:, :, :, None],
        (_NW, nm, _K, _DW))

    (m1,) = _tc_mm(n, d)(x, W1)
    degp = _deg_kernel(n_pad, ne0, ne1, rpt)(idxp, ewb)
    dis, h1p = _tc_prep(n, d)(m1, degp)

    agg = _agg_kernel(d, n_pad, ne0, ne1, rpt)
    a1 = agg(h1p, idxp, ewb)
    (h2p,) = _tc_mid(n, d)(a1, h1p, dis, b1.reshape(1, d), W2)
    a2 = agg(h2p, idxp, ewb)

    (out,) = _tc_head(n, d, nb, dout)(
        a2, h2p, dis, b2.reshape(1, d),
        batch.reshape(1, n).astype(jnp.int32),
        jnp.asarray(batch_size, jnp.int32).reshape(1),
        u, Wh1[:d], Wh1[d:], bh1.reshape(1, d), Wh2, bh2.reshape(1, dout))
    return out


# speedup vs baseline: 1.0926x; 1.0022x over previous
import jax
import jax.numpy as jnp
from jax import lax
from jax.experimental import pallas as pl
from jax.experimental.pallas import tpu as pltpu
from jax.experimental.pallas import tpu_sc as plsc

_NC = 2
_NS = 16
_NW = _NC * _NS
_L = 16
_K = 80
_DW = 16
_FRAC0 = 0.50


def _sc_mesh():
    return plsc.VectorSubcoreMesh(core_axis_name="c", subcore_axis_name="s",
                                  num_cores=_NC, num_subcores=_NS)


def _scale_rows(rows, ewb_v, d):
    def grp(g, carry):
        for l in range(_L):
            e = g * _L + l
            w = ewb_v[e, :]
            for j in range(d // _L):
                sl = pl.ds(j * _L, _L)
                rows[e, sl] = rows[e, sl] * w
        return carry

    lax.fori_loop(0, _K // _L, grp, 0)


def _zero_stripe(rows, acc, row0, nrep):
    zv = jnp.zeros((_L,), jnp.float32)

    def grp(g, carry):
        for l in range(_L):
            e = g * _L + l
            for j in range(128 // _L):
                rows[e, pl.ds(j * _L, _L)] = zv
        return carry

    lax.fori_loop(0, _K // _L, grp, 0)
    for i in range(nrep):
        pltpu.sync_copy(rows, acc.at[pl.ds(row0 + i * _K, _K)])


def _fill_rows(rows, ewb_v):
    def grp(g, carry):
        for l in range(_L):
            e = g * _L + l
            w = ewb_v[e, :]
            for j in range(128 // _L):
                rows[e, pl.ds(j * _L, _L)] = w
        return carry

    lax.fori_loop(0, _K // _L, grp, 0)


def _deg_kernel(n_pad, ne0, ne1, rpt):
    def body(idx_hbm, ewb_hbm, out_hbm,
             idx0, idx1, ewb0, ewb1, rows0, rows1, acc,
             si0, si1, se0, se1, ss0):
        c = lax.axis_index("c")
        s = lax.axis_index("s")
        wid = c * _NS + s
        row0 = s * rpt
        _zero_stripe(rows0, acc, row0, rpt // _K)
        plsc.subcore_barrier()

        pltpu.sync_copy(idx_hbm.at[wid, 0], idx0)
        pltpu.sync_copy(ewb_hbm.at[wid, 0], ewb0)

        def pair(i, carry):
            a = 2 * i
            b = a + 1
            pltpu.async_copy(idx_hbm.at[wid, b], idx1, si1)
            pltpu.async_copy(ewb_hbm.at[wid, b], ewb1, se1)
            _fill_rows(rows0, ewb0)
            pltpu.make_async_copy(idx_hbm.at[wid, b], idx1, si1).wait()
            pltpu.make_async_copy(ewb_hbm.at[wid, b], ewb1, se1).wait()
            pltpu.async_copy(rows0, acc.at[idx0.at[1]], ss0, add=True)
            _fill_rows(rows1, ewb1)
            pltpu.make_async_copy(rows0, acc.at[idx0.at[1]], ss0).wait()
            pltpu.async_copy(idx_hbm.at[wid, a + 2], idx0, si0)
            pltpu.async_copy(ewb_hbm.at[wid, a + 2], ewb0, se0)
            pltpu.sync_copy(rows1, acc.at[idx1.at[1]], add=True)
            pltpu.make_async_copy(idx_hbm.at[wid, a + 2], idx0, si0).wait()
            pltpu.make_async_copy(ewb_hbm.at[wid, a + 2], ewb0, se0).wait()
            return carry

        trip = lax.select(c == 0, ne0 // 2, ne1 // 2)
        lax.fori_loop(0, trip, pair, 0)
        plsc.subcore_barrier()
        pltpu.sync_copy(acc.at[pl.ds(row0, rpt)], out_hbm.at[c, pl.ds(row0, rpt)])

    return pl.kernel(
        body,
        out_type=jax.ShapeDtypeStruct((_NC, n_pad, 128), jnp.float32),
        mesh=_sc_mesh(),
        scratch_types=[
            pltpu.VMEM((2, _K), jnp.int32),
            pltpu.VMEM((2, _K), jnp.int32),
            pltpu.VMEM((_K, _DW), jnp.float32),
            pltpu.VMEM((_K, _DW), jnp.float32),
            pltpu.VMEM((_K, 128), jnp.float32),
            pltpu.VMEM((_K, 128), jnp.float32),
            pltpu.VMEM_SHARED((n_pad, 128), jnp.float32),
            pltpu.SemaphoreType.DMA,
            pltpu.SemaphoreType.DMA,
            pltpu.SemaphoreType.DMA,
            pltpu.SemaphoreType.DMA,
            pltpu.SemaphoreType.DMA,
        ],
    )


def _agg_kernel(d, n_pad, ne0, ne1, rpt):
    def body(h_hbm, idx_hbm, ewb_hbm, out_hbm,
             idx0, idx1, ewb0, ewb1, rows0, rows1, acc,
             sg0, sg1, si0, si1, se0, se1, ss0):
        c = lax.axis_index("c")
        s = lax.axis_index("s")
        wid = c * _NS + s
        row0 = s * rpt
        _zero_stripe(rows0, acc, row0, rpt // _K)
        plsc.subcore_barrier()

        pltpu.sync_copy(idx_hbm.at[wid, 0], idx0)
        pltpu.sync_copy(ewb_hbm.at[wid, 0], ewb0)
        pltpu.async_copy(h_hbm.at[idx0.at[0]], rows0, sg0)

        def pair(i, carry):
            a = 2 * i
            b = a + 1
            pltpu.async_copy(idx_hbm.at[wid, b], idx1, si1)
            pltpu.async_copy(ewb_hbm.at[wid, b], ewb1, se1)
            pltpu.make_async_copy(h_hbm.at[idx0.at[0]], rows0, sg0).wait()
            _scale_rows(rows0, ewb0, d)
            pltpu.make_async_copy(idx_hbm.at[wid, b], idx1, si1).wait()
            pltpu.make_async_copy(ewb_hbm.at[wid, b], ewb1, se1).wait()
            pltpu.async_copy(h_hbm.at[idx1.at[0]], rows1, sg1)
            pltpu.async_copy(rows0, acc.at[idx0.at[1]], ss0, add=True)
            pltpu.make_async_copy(h_hbm.at[idx1.at[0]], rows1, sg1).wait()
            _scale_rows(rows1, ewb1, d)
            pltpu.make_async_copy(rows0, acc.at[idx0.at[1]], ss0).wait()
            pltpu.async_copy(idx_hbm.at[wid, a + 2], idx0, si0)
            pltpu.async_copy(ewb_hbm.at[wid, a + 2], ewb0, se0)
            pltpu.sync_copy(rows1, acc.at[idx1.at[1]], add=True)
            pltpu.make_async_copy(idx_hbm.at[wid, a + 2], idx0, si0).wait()
            pltpu.make_async_copy(ewb_hbm.at[wid, a + 2], ewb0, se0).wait()
            pltpu.async_copy(h_hbm.at[idx0.at[0]], rows0, sg0)
            return carry

        trip = lax.select(c == 0, ne0 // 2, ne1 // 2)
        lax.fori_loop(0, trip, pair, 0)
        pltpu.make_async_copy(h_hbm.at[idx0.at[0]], rows0, sg0).wait()
        plsc.subcore_barrier()
        pltpu.sync_copy(acc.at[pl.ds(row0, rpt)], out_hbm.at[c, pl.ds(row0, rpt)])

    return pl.kernel(
        body,
        out_type=jax.ShapeDtypeStruct((_NC, n_pad, d), jnp.float32),
        mesh=_sc_mesh(),
        scratch_types=[
            pltpu.VMEM((2, _K), jnp.int32),
            pltpu.VMEM((2, _K), jnp.int32),
            pltpu.VMEM((_K, _DW), jnp.float32),
            pltpu.VMEM((_K, _DW), jnp.float32),
            pltpu.VMEM((_K, d), jnp.float32),
            pltpu.VMEM((_K, d), jnp.float32),
            pltpu.VMEM_SHARED((n_pad, d), jnp.float32),
            pltpu.SemaphoreType.DMA,
            pltpu.SemaphoreType.DMA,
            pltpu.SemaphoreType.DMA,
            pltpu.SemaphoreType.DMA,
            pltpu.SemaphoreType.DMA,
            pltpu.SemaphoreType.DMA,
            pltpu.SemaphoreType.DMA,
        ],
    )


def _tc_prep(n, d):
    def body(x_ref, w_ref, dp_ref, dis_ref, hp_ref):
        deg = dp_ref[0, :n, 0:1] + dp_ref[1, :n, 0:1] + 1.0
        dis = lax.rsqrt(deg)
        dis_ref[...] = dis
        h = jnp.dot(x_ref[...], w_ref[...], preferred_element_type=jnp.float32)
        hp_ref[...] = h * dis

    return pl.pallas_call(
        body,
        out_shape=[
            jax.ShapeDtypeStruct((n, 1), jnp.float32),
            jax.ShapeDtypeStruct((n, d), jnp.float32),
        ],
    )


def _tc_mid(n, d):
    def body(ap_ref, hp_ref, dis_ref, b_ref, w_ref, out_ref):
        dis = dis_ref[...]
        pre = (ap_ref[0, :n, :] + ap_ref[1, :n, :] + hp_ref[...]) * dis + b_ref[...]
        h = jnp.maximum(pre, 0.0)
        out_ref[...] = jnp.dot(h, w_ref[...], preferred_element_type=jnp.float32) * dis

    return pl.pallas_call(
        body,
        out_shape=[jax.ShapeDtypeStruct((n, d), jnp.float32)],
    )


def _tc_head(n, d, nb, dout):
    def body(ap_ref, hp_ref, dis_ref, b_ref, batch_ref, bs_ref, u_ref,
             wha_ref, whb_ref, bh1_ref, wh2_ref, bh2_ref, out_ref):
        dis = dis_ref[...]
        h2 = jnp.maximum(
            (ap_ref[0, :n, :] + ap_ref[1, :n, :] + hp_ref[...]) * dis + b_ref[...],
            0.0)
        seg = lax.broadcasted_iota(jnp.int32, (nb, n), 0)
        onehot = (seg == batch_ref[...]).astype(jnp.float32)
        sums = jnp.dot(onehot, h2, preferred_element_type=jnp.float32)
        cnts = jnp.sum(onehot, axis=1, keepdims=True)
        gem = sums / jnp.maximum(cnts, 1.0)
        valid = lax.broadcasted_iota(jnp.int32, (nb, 1), 0) < bs_ref[0]
        gem = jnp.where(valid, gem, 0.0)
        z = jnp.maximum(
            jnp.dot(gem, wha_ref[...], preferred_element_type=jnp.float32)
            + jnp.dot(u_ref[...], whb_ref[...], preferred_element_type=jnp.float32)
            + bh1_ref[...], 0.0)
        out_ref[...] = (jnp.dot(z, wh2_ref[...], preferred_element_type=jnp.float32)
                        + bh2_ref[...])

    nargs = 12
    specs = [pl.BlockSpec(memory_space=pltpu.VMEM) for _ in range(nargs)]
    specs[5] = pl.BlockSpec(memory_space=pltpu.SMEM)
    return pl.pallas_call(
        body,
        in_specs=specs,
        out_specs=[pl.BlockSpec(memory_space=pltpu.VMEM)],
        out_shape=[jax.ShapeDtypeStruct((nb, dout), jnp.float32)],
    )


def kernel(x, edge_index, u, edge_weight, batch, batch_size,
           W1, b1, W2, b2, Wh1, bh1, Wh2, bh2):
    n, _ = x.shape
    d = W1.shape[1]
    e = edge_weight.shape[0]
    nb, dg = u.shape
    dout = Wh2.shape[1]

    grp0 = _NS * _K
    ne0 = -(-int(e * _FRAC0) // grp0)
    ne0 = ne0 + (ne0 % 2)
    e0 = min(e, grp0 * ne0)
    ne1 = -(-(e - e0) // grp0)
    ne1 = ne1 + (ne1 % 2)
    nm = max(ne0, ne1) + 2
    rpt = -(-(-(-n // _NS)) // _K) * _K
    n_pad = rpt * _NS

    def split(arr, dtype):
        a0 = arr[:e0].reshape(_NS, ne0, _K)
        a0 = jnp.pad(a0, ((0, 0), (0, nm - ne0), (0, 0)))
        a1 = jnp.pad(arr[e0:], (0, grp0 * ne1 - (e - e0)))
        a1 = jnp.pad(a1.reshape(_NS, ne1, _K), ((0, 0), (0, nm - ne1), (0, 0)))
        return jnp.concatenate([a0, a1], axis=0).astype(dtype)

    srcp = split(edge_index[0], jnp.int32)[:, :, None, :]
    dstp = split(edge_index[1], jnp.int32)[:, :, None, :]
    idxp = jnp.concatenate([srcp, dstp], axis=2)
    ewb = jnp.broadcast_to(
        split(edge_weight, jnp.float32)[:, :, :, None],
        (_NW, nm, _K, _DW))

    degp = _deg_kernel(n_pad, ne0, ne1, rpt)(idxp, ewb)
    dis, h1p = _tc_prep(n, d)(x, W1, degp)

    agg = _agg_kernel(d, n_pad, ne0, ne1, rpt)
    a1 = agg(h1p, idxp, ewb)
    (h2p,) = _tc_mid(n, d)(a1, h1p, dis, b1.reshape(1, d), W2)
    a2 = agg(h2p, idxp, ewb)

    (out,) = _tc_head(n, d, nb, dout)(
        a2, h2p, dis, b2.reshape(1, d),
        batch.reshape(1, n).astype(jnp.int32),
        jnp.asarray(batch_size, jnp.int32).reshape(1),
        u, Wh1[:d], Wh1[d:], bh1.reshape(1, d), Wh2, bh2.reshape(1, dout))
    return out
